# baseline (speedup 1.0000x reference)
import functools

import jax
import jax.numpy as jnp
from jax import lax
from jax.experimental import pallas as pl
from jax.experimental.pallas import tpu as pltpu
from jax.experimental.pallas import tpu_sc as plsc

F32 = jnp.float32
NC, NS = 2, 16
NW = NC * NS
CHUNK = 128


def _tc_matmul(xp, W, br=2048):
    Np, D = xp.shape
    C = W.shape[1]

    def body(x_ref, w_ref, o_ref):
        o_ref[...] = jnp.dot(x_ref[...], w_ref[...], preferred_element_type=F32)

    return pl.pallas_call(
        body,
        grid=(Np // br,),
        in_specs=[pl.BlockSpec((br, D), lambda i: (i, 0)),
                  pl.BlockSpec((D, C), lambda i: (0, 0))],
        out_specs=pl.BlockSpec((br, C), lambda i: (i, 0)),
        out_shape=jax.ShapeDtypeStruct((Np, C), F32),
    )(xp, W)


def _tc_norm_matmul(pa, pb, ca, cb, y, b, W, br=2048):
    Np, C = y.shape
    C2 = W.shape[1]

    def body(pa_ref, pb_ref, ca_ref, cb_ref, y_ref, b_ref, w_ref, o_ref):
        s = pa_ref[...] + pb_ref[...] + y_ref[...]
        cnt = ca_ref[...] + cb_ref[...] + 1.0
        if C == 16:
            cw = cnt
        else:
            cw = jnp.broadcast_to(cnt[:, :1], (br, C))
        h = jnp.maximum(s / cw + b_ref[...], 0.0)
        o_ref[...] = jnp.dot(h, w_ref[...], preferred_element_type=F32)

    return pl.pallas_call(
        body,
        grid=(Np // br,),
        in_specs=[pl.BlockSpec((br, C), lambda i: (i, 0)),
                  pl.BlockSpec((br, C), lambda i: (i, 0)),
                  pl.BlockSpec((br, 16), lambda i: (i, 0)),
                  pl.BlockSpec((br, 16), lambda i: (i, 0)),
                  pl.BlockSpec((br, C), lambda i: (i, 0)),
                  pl.BlockSpec((1, C), lambda i: (0, 0)),
                  pl.BlockSpec((C, C2), lambda i: (0, 0))],
        out_specs=pl.BlockSpec((br, C2), lambda i: (i, 0)),
        out_shape=jax.ShapeDtypeStruct((Np, C2), F32),
    )(pa, pb, ca, cb, y, b, W)


def _tc_head(pa, pb, ca, cb, y, b, lw1, lb1, lw2, lb2, lw3, lb3, lw4, lb4,
             ow, ob, br=2048):
    Np, C = y.shape

    def body(pa_ref, pb_ref, ca_ref, cb_ref, y_ref, b_ref,
             w1, v1, w2, v2, w3, v3, w4, v4, wo, vo, o_ref):
        s = pa_ref[...] + pb_ref[...] + y_ref[...]
        cnt = ca_ref[...] + cb_ref[...] + 1.0
        cw = jnp.broadcast_to(cnt[:, :1], (br, C))
        h = jnp.maximum(s / cw + b_ref[...], 0.0)
        h = jnp.maximum(jnp.dot(h, w1[...], preferred_element_type=F32) + v1[...], 0.0)
        h = jnp.maximum(jnp.dot(h, w2[...], preferred_element_type=F32) + v2[...], 0.0)
        h = jnp.maximum(jnp.dot(h, w3[...], preferred_element_type=F32) + v3[...], 0.0)
        h = jnp.maximum(jnp.dot(h, w4[...], preferred_element_type=F32) + v4[...], 0.0)
        o = jnp.dot(h, wo[...], preferred_element_type=F32) + vo[...]
        o_ref[...] = jax.nn.sigmoid(o)

    def mat(a):
        return pl.BlockSpec(a.shape, lambda i: (0, 0))

    return pl.pallas_call(
        body,
        grid=(Np // br,),
        in_specs=[pl.BlockSpec((br, C), lambda i: (i, 0)),
                  pl.BlockSpec((br, C), lambda i: (i, 0)),
                  pl.BlockSpec((br, 16), lambda i: (i, 0)),
                  pl.BlockSpec((br, 16), lambda i: (i, 0)),
                  pl.BlockSpec((br, C), lambda i: (i, 0)),
                  pl.BlockSpec((1, C), lambda i: (0, 0)),
                  mat(lw1), mat(lb1), mat(lw2), mat(lb2),
                  mat(lw3), mat(lb3), mat(lw4), mat(lb4),
                  mat(ow), mat(ob)],
        out_specs=pl.BlockSpec((br, 1), lambda i: (i, 0)),
        out_shape=jax.ShapeDtypeStruct((Np, 1), F32),
    )(pa, pb, ca, cb, y, b, lw1, lb1, lw2, lb2, lw3, lb3, lw4, lb4, ow, ob)


def _sc_body(with_cnt, *refs):
    if with_cnt:
        (y_ref, src_ref, dst_ref, out_ref, cnt_out_ref,
         acc_sh, cnt_sh, src_v, dst_v, rows_v, zero_v, ones_v,
         *sems) = refs
        sems_g, sems_s, sems_c = sems[0:4], sems[4:8], sems[8:12]
    else:
        (y_ref, src_ref, dst_ref, out_ref,
         acc_sh, src_v, dst_v, rows_v, zero_v, *sems) = refs
        sems_g, sems_s = sems[0:4], sems[4:8]

    Np, C = acc_sh.shape
    NCH = src_v.shape[0] - 2
    rt = Np // NS
    c = lax.axis_index("c")
    s = lax.axis_index("s")
    wid = c * NS + s

    z16 = jnp.zeros((16,), F32)

    def zrow(i, _):
        for k in range(C // 16):
            zero_v[i, pl.ds(k * 16, 16)] = z16
        return 0

    lax.fori_loop(0, CHUNK, zrow, 0)
    if with_cnt:
        o16 = jnp.ones((16,), F32)

        def orow(i, _):
            ones_v[i, pl.ds(0, 16)] = o16
            return 0

        lax.fori_loop(0, CHUNK, orow, 0)

    for q in range(rt // CHUNK):
        off = s * rt + q * CHUNK
        pltpu.sync_copy(zero_v, acc_sh.at[pl.ds(off, CHUNK)])
        if with_cnt:
            pltpu.sync_copy(zero_v, cnt_sh.at[pl.ds(off, CHUNK)])
    plsc.subcore_barrier()

    pltpu.sync_copy(src_ref.at[wid], src_v.at[pl.ds(0, NCH)])
    pltpu.sync_copy(dst_ref.at[wid], dst_v.at[pl.ds(0, NCH)])
    sent16 = jnp.full((16,), Np - 1, jnp.int32)
    for p in range(2):
        for k in range(CHUNK // 16):
            src_v[NCH + p, pl.ds(k * 16, 16)] = sent16

    def g_start(j, b):
        pltpu.async_copy(y_ref.at[src_v.at[j]], rows_v.at[b], sems_g[b])

    def g_wait(j, b):
        pltpu.make_async_copy(y_ref.at[src_v.at[j]], rows_v.at[b],
                              sems_g[b]).wait()

    def s_start(j, b):
        pltpu.async_copy(rows_v.at[b], acc_sh.at[dst_v.at[j]], sems_s[b],
                         add=True)
        if with_cnt:
            pltpu.async_copy(ones_v, cnt_sh.at[dst_v.at[j]], sems_c[b],
                             add=True)

    def s_wait(j, b):
        pltpu.make_async_copy(rows_v.at[b], acc_sh.at[dst_v.at[j]],
                              sems_s[b]).wait()
        if with_cnt:
            pltpu.make_async_copy(ones_v, cnt_sh.at[dst_v.at[j]],
                                  sems_c[b]).wait()

    g_start(0, 0)
    g_start(1, 1)
    g_wait(0, 0); s_start(0, 0); g_start(2, 2)
    g_wait(1, 1); s_start(1, 1); g_start(3, 3)
    g_wait(2, 2); s_start(2, 2); s_wait(0, 0); g_start(4, 0)
    g_wait(3, 3); s_start(3, 3); s_wait(1, 1); g_start(5, 1)

    def chunk_body(j4, _):
        jb = 4 + 4 * j4
        for b in range(4):
            j = jb + b
            g_wait(j, b)
            s_start(j, b)
            s_wait(j - 2, (b + 2) % 4)
            g_start(j + 2, (b + 2) % 4)
        return 0

    lax.fori_loop(0, (NCH - 4) // 4, chunk_body, 0)
    s_wait(NCH - 2, (NCH - 2) % 4)
    s_wait(NCH - 1, (NCH - 1) % 4)
    g_wait(NCH, NCH % 4)
    g_wait(NCH + 1, (NCH + 1) % 4)
    plsc.subcore_barrier()

    pltpu.sync_copy(acc_sh.at[pl.ds(s * rt, rt)], out_ref.at[c, pl.ds(s * rt, rt)])
    if with_cnt:
        pltpu.sync_copy(cnt_sh.at[pl.ds(s * rt, rt)],
                        cnt_out_ref.at[c, pl.ds(s * rt, rt)])


def _sc_aggregate(y, src3, dst3, with_cnt):
    Np, C = y.shape
    NCH = src3.shape[1]
    mesh = plsc.VectorSubcoreMesh(core_axis_name="c", subcore_axis_name="s",
                                  num_cores=NC, num_subcores=NS)
    out_type = [jax.ShapeDtypeStruct((NC, Np, C), F32)]
    assert NCH % 4 == 0 and NCH >= 8
    scratch = [
        pltpu.VMEM_SHARED((Np, C), F32),
        pltpu.VMEM((NCH + 2, CHUNK), jnp.int32),
        pltpu.VMEM((NCH + 2, CHUNK), jnp.int32),
        pltpu.VMEM((4, CHUNK, C), F32),
        pltpu.VMEM((CHUNK, C), F32),
    ] + [pltpu.SemaphoreType.DMA] * 8
    if with_cnt:
        out_type.append(jax.ShapeDtypeStruct((NC, Np, 16), F32))
        scratch.insert(1, pltpu.VMEM_SHARED((Np, 16), F32))
        scratch.insert(6, pltpu.VMEM((CHUNK, 16), F32))
        scratch += [pltpu.SemaphoreType.DMA] * 4
    fn = pl.kernel(functools.partial(_sc_body, with_cnt),
                   out_type=tuple(out_type), mesh=mesh,
                   scratch_types=tuple(scratch),
                   compiler_params=pltpu.CompilerParams(use_tc_tiling_on_sc=False))
    res = fn(y, src3, dst3)
    return res if isinstance(res, (tuple, list)) else (res,)


def kernel(x, edge_index, W1, u1, c1, b1, W2, u2, c2, b2, W3, u3, c3, b3,
           lw1, lb1, lw2, lb2, lw3, lb3, lw4, lb4, ow, ob):
    N, D = x.shape
    E = edge_index.shape[1]
    Np = 10240
    epw = -(-E // NW)
    nch = -(-epw // CHUNK)
    nch = -(-nch // 4) * 4
    ep = NW * nch * CHUNK

    xpad = jnp.zeros((Np, D), F32).at[:N].set(x)
    sentinel = jnp.int32(Np - 1)
    src3 = jnp.full((ep,), sentinel, jnp.int32).at[:E].set(edge_index[0]).reshape(NW, nch, CHUNK)
    dst3 = jnp.full((ep,), sentinel, jnp.int32).at[:E].set(edge_index[1]).reshape(NW, nch, CHUNK)

    b1r = b1.reshape(1, -1)
    b2r = b2.reshape(1, -1)
    b3r = b3.reshape(1, -1)

    y1 = _tc_matmul(xpad, W1)
    s1, cnt = _sc_aggregate(y1, src3, dst3, with_cnt=True)
    y2 = _tc_norm_matmul(s1[0], s1[1], cnt[0], cnt[1], y1, b1r, W2)
    (s2,) = _sc_aggregate(y2, src3, dst3, with_cnt=False)
    y3 = _tc_norm_matmul(s2[0], s2[1], cnt[0], cnt[1], y2, b2r, W3)
    (s3,) = _sc_aggregate(y3, src3, dst3, with_cnt=False)
    out = _tc_head(s3[0], s3[1], cnt[0], cnt[1], y3, b3r,
                   lw1, lb1.reshape(1, -1), lw2, lb2.reshape(1, -1),
                   lw3, lb3.reshape(1, -1), lw4, lb4.reshape(1, -1),
                   ow, ob.reshape(1, -1))
    return out[:N]


# speedup vs baseline: 1.1687x; 1.1687x over previous
import functools

import jax
import jax.numpy as jnp
from jax import lax
from jax.experimental import pallas as pl
from jax.experimental.pallas import tpu as pltpu
from jax.experimental.pallas import tpu_sc as plsc

F32 = jnp.float32
NC, NS = 2, 16
NW = NC * NS
CHUNK = 128


def _tc_matmul(xp, W, br=2048):
    Np, D = xp.shape
    C = W.shape[1]

    def body(x_ref, w_ref, o_ref):
        o_ref[...] = jnp.dot(x_ref[...], w_ref[...], preferred_element_type=F32)

    return pl.pallas_call(
        body,
        grid=(Np // br,),
        in_specs=[pl.BlockSpec((br, D), lambda i: (i, 0)),
                  pl.BlockSpec((D, C), lambda i: (0, 0))],
        out_specs=pl.BlockSpec((br, C), lambda i: (i, 0)),
        out_shape=jax.ShapeDtypeStruct((Np, C), F32),
    )(xp, W)


def _tc_norm_matmul(pa, pb, ca, cb, y, b, W, br=2048):
    Np, C = y.shape
    C2 = W.shape[1]

    def body(pa_ref, pb_ref, ca_ref, cb_ref, y_ref, b_ref, w_ref, o_ref):
        s = pa_ref[...] + pb_ref[...] + y_ref[...]
        cnt = ca_ref[...] + cb_ref[...] + 1.0
        if C == 16:
            cw = cnt
        else:
            cw = jnp.broadcast_to(cnt[:, :1], (br, C))
        h = jnp.maximum(s / cw + b_ref[...], 0.0)
        o_ref[...] = jnp.dot(h, w_ref[...], preferred_element_type=F32)

    return pl.pallas_call(
        body,
        grid=(Np // br,),
        in_specs=[pl.BlockSpec((br, C), lambda i: (i, 0)),
                  pl.BlockSpec((br, C), lambda i: (i, 0)),
                  pl.BlockSpec((br, 16), lambda i: (i, 0)),
                  pl.BlockSpec((br, 16), lambda i: (i, 0)),
                  pl.BlockSpec((br, C), lambda i: (i, 0)),
                  pl.BlockSpec((1, C), lambda i: (0, 0)),
                  pl.BlockSpec((C, C2), lambda i: (0, 0))],
        out_specs=pl.BlockSpec((br, C2), lambda i: (i, 0)),
        out_shape=jax.ShapeDtypeStruct((Np, C2), F32),
    )(pa, pb, ca, cb, y, b, W)


def _tc_head(pa, pb, ca, cb, y, b, lw1, lb1, lw2, lb2, lw3, lb3, lw4, lb4,
             ow, ob, br=2048):
    Np, C = y.shape

    def body(pa_ref, pb_ref, ca_ref, cb_ref, y_ref, b_ref,
             w1, v1, w2, v2, w3, v3, w4, v4, wo, vo, o_ref):
        s = pa_ref[...] + pb_ref[...] + y_ref[...]
        cnt = ca_ref[...] + cb_ref[...] + 1.0
        cw = jnp.broadcast_to(cnt[:, :1], (br, C))
        h = jnp.maximum(s / cw + b_ref[...], 0.0)
        h = jnp.maximum(jnp.dot(h, w1[...], preferred_element_type=F32) + v1[...], 0.0)
        h = jnp.maximum(jnp.dot(h, w2[...], preferred_element_type=F32) + v2[...], 0.0)
        h = jnp.maximum(jnp.dot(h, w3[...], preferred_element_type=F32) + v3[...], 0.0)
        h = jnp.maximum(jnp.dot(h, w4[...], preferred_element_type=F32) + v4[...], 0.0)
        o = jnp.dot(h, wo[...], preferred_element_type=F32) + vo[...]
        o_ref[...] = jax.nn.sigmoid(o)

    def mat(a):
        return pl.BlockSpec(a.shape, lambda i: (0, 0))

    return pl.pallas_call(
        body,
        grid=(Np // br,),
        in_specs=[pl.BlockSpec((br, C), lambda i: (i, 0)),
                  pl.BlockSpec((br, C), lambda i: (i, 0)),
                  pl.BlockSpec((br, 16), lambda i: (i, 0)),
                  pl.BlockSpec((br, 16), lambda i: (i, 0)),
                  pl.BlockSpec((br, C), lambda i: (i, 0)),
                  pl.BlockSpec((1, C), lambda i: (0, 0)),
                  mat(lw1), mat(lb1), mat(lw2), mat(lb2),
                  mat(lw3), mat(lb3), mat(lw4), mat(lb4),
                  mat(ow), mat(ob)],
        out_specs=pl.BlockSpec((br, 1), lambda i: (i, 0)),
        out_shape=jax.ShapeDtypeStruct((Np, 1), F32),
    )(pa, pb, ca, cb, y, b, lw1, lb1, lw2, lb2, lw3, lb3, lw4, lb4, ow, ob)


def _sc_body(with_cnt, *refs):
    if with_cnt:
        (y_ref, src_ref, dst_ref, out_ref, cnt_out_ref,
         acc_sh, cnt_sh, src_v, dst_v, rows_v, zero_v, ones_v,
         *sems_g) = refs
    else:
        (y_ref, src_ref, dst_ref, out_ref,
         acc_sh, src_v, dst_v, rows_v, zero_v, *sems_g) = refs

    Np, C = acc_sh.shape
    NCH = src_v.shape[0] - 2
    rt = Np // NS
    c = lax.axis_index("c")
    s = lax.axis_index("s")
    wid = c * NS + s

    z16 = jnp.zeros((16,), F32)

    def zrow(i, _):
        for k in range(C // 16):
            zero_v[i, pl.ds(k * 16, 16)] = z16
        return 0

    lax.fori_loop(0, CHUNK, zrow, 0)
    if with_cnt:
        o16 = jnp.ones((16,), F32)

        def orow(i, _):
            ones_v[i, pl.ds(0, 16)] = o16
            return 0

        lax.fori_loop(0, CHUNK, orow, 0)

    for q in range(rt // CHUNK):
        off = s * rt + q * CHUNK
        pltpu.sync_copy(zero_v, acc_sh.at[pl.ds(off, CHUNK)])
        if with_cnt:
            pltpu.sync_copy(zero_v, cnt_sh.at[pl.ds(off, CHUNK)])
    plsc.subcore_barrier()

    pltpu.sync_copy(src_ref.at[wid], src_v.at[pl.ds(0, NCH)])
    pltpu.sync_copy(dst_ref.at[wid], dst_v.at[pl.ds(0, NCH)])
    sent16 = jnp.full((16,), Np - 1, jnp.int32)
    for p in range(2):
        for k in range(CHUNK // 16):
            src_v[NCH + p, pl.ds(k * 16, 16)] = sent16

    def g_start(j, b):
        pltpu.async_copy(y_ref.at[src_v.at[j]], rows_v.at[b], sems_g[b])

    def g_wait(j, b):
        pltpu.make_async_copy(y_ref.at[src_v.at[j]], rows_v.at[b],
                              sems_g[b]).wait()

    g_start(0, 0)

    def chunk_body(j2, _):
        for b in range(2):
            j = 2 * j2 + b
            g_start(j + 1, 1 - b)
            g_wait(j, b)
            pltpu.sync_copy(rows_v.at[b], acc_sh.at[dst_v.at[j]], add=True)
            if with_cnt:
                pltpu.sync_copy(ones_v, cnt_sh.at[dst_v.at[j]], add=True)
        return 0

    lax.fori_loop(0, NCH // 2, chunk_body, 0)
    g_wait(NCH, 0)
    plsc.subcore_barrier()

    pltpu.sync_copy(acc_sh.at[pl.ds(s * rt, rt)], out_ref.at[c, pl.ds(s * rt, rt)])
    if with_cnt:
        pltpu.sync_copy(cnt_sh.at[pl.ds(s * rt, rt)],
                        cnt_out_ref.at[c, pl.ds(s * rt, rt)])


def _sc_aggregate(y, src3, dst3, with_cnt):
    Np, C = y.shape
    NCH = src3.shape[1]
    mesh = plsc.VectorSubcoreMesh(core_axis_name="c", subcore_axis_name="s",
                                  num_cores=NC, num_subcores=NS)
    out_type = [jax.ShapeDtypeStruct((NC, Np, C), F32)]
    assert NCH % 4 == 0 and NCH >= 8
    scratch = [
        pltpu.VMEM_SHARED((Np, C), F32),
        pltpu.VMEM((NCH + 2, CHUNK), jnp.int32),
        pltpu.VMEM((NCH + 2, CHUNK), jnp.int32),
        pltpu.VMEM((2, CHUNK, C), F32),
        pltpu.VMEM((CHUNK, C), F32),
    ] + [pltpu.SemaphoreType.DMA] * 2
    if with_cnt:
        out_type.append(jax.ShapeDtypeStruct((NC, Np, 16), F32))
        scratch.insert(1, pltpu.VMEM_SHARED((Np, 16), F32))
        scratch.insert(6, pltpu.VMEM((CHUNK, 16), F32))
    fn = pl.kernel(functools.partial(_sc_body, with_cnt),
                   out_type=tuple(out_type), mesh=mesh,
                   scratch_types=tuple(scratch),
                   compiler_params=pltpu.CompilerParams(use_tc_tiling_on_sc=False))
    res = fn(y, src3, dst3)
    return res if isinstance(res, (tuple, list)) else (res,)


def kernel(x, edge_index, W1, u1, c1, b1, W2, u2, c2, b2, W3, u3, c3, b3,
           lw1, lb1, lw2, lb2, lw3, lb3, lw4, lb4, ow, ob):
    N, D = x.shape
    E = edge_index.shape[1]
    Np = 10240
    epw = -(-E // NW)
    nch = -(-epw // CHUNK)
    nch = -(-nch // 4) * 4
    ep = NW * nch * CHUNK

    xpad = jnp.zeros((Np, D), F32).at[:N].set(x)
    sentinel = jnp.int32(Np - 1)
    src3 = jnp.full((ep,), sentinel, jnp.int32).at[:E].set(edge_index[0]).reshape(NW, nch, CHUNK)
    dst3 = jnp.full((ep,), sentinel, jnp.int32).at[:E].set(edge_index[1]).reshape(NW, nch, CHUNK)

    b1r = b1.reshape(1, -1)
    b2r = b2.reshape(1, -1)
    b3r = b3.reshape(1, -1)

    y1 = _tc_matmul(xpad, W1)
    s1, cnt = _sc_aggregate(y1, src3, dst3, with_cnt=True)
    y2 = _tc_norm_matmul(s1[0], s1[1], cnt[0], cnt[1], y1, b1r, W2)
    (s2,) = _sc_aggregate(y2, src3, dst3, with_cnt=False)
    y3 = _tc_norm_matmul(s2[0], s2[1], cnt[0], cnt[1], y2, b2r, W3)
    (s3,) = _sc_aggregate(y3, src3, dst3, with_cnt=False)
    out = _tc_head(s3[0], s3[1], cnt[0], cnt[1], y3, b3r,
                   lw1, lb1.reshape(1, -1), lw2, lb2.reshape(1, -1),
                   lw3, lb3.reshape(1, -1), lw4, lb4.reshape(1, -1),
                   ow, ob.reshape(1, -1))
    return out[:N]


# speedup vs baseline: 2.5240x; 2.1597x over previous
import functools

import jax
import jax.numpy as jnp
from jax import lax
from jax.experimental import pallas as pl
from jax.experimental.pallas import tpu as pltpu
from jax.experimental.pallas import tpu_sc as plsc

F32 = jnp.float32
NC, NS = 2, 16
NW = NC * NS
CHUNK = 128


def _tc_matmul(xp, W, br=2048):
    Np, D = xp.shape
    C = W.shape[1]

    def body(x_ref, w_ref, o_ref):
        o_ref[...] = jnp.dot(x_ref[...], w_ref[...], preferred_element_type=F32)

    return pl.pallas_call(
        body,
        grid=(Np // br,),
        in_specs=[pl.BlockSpec((br, D), lambda i: (i, 0)),
                  pl.BlockSpec((D, C), lambda i: (0, 0))],
        out_specs=pl.BlockSpec((br, C), lambda i: (i, 0)),
        out_shape=jax.ShapeDtypeStruct((Np, C), F32),
    )(xp, W)


def _tc_norm_matmul(pa, pb, ca, cb, y, b, W, br=2048):
    Np, C = y.shape
    C2 = W.shape[1]

    def body(pa_ref, pb_ref, ca_ref, cb_ref, y_ref, b_ref, w_ref, o_ref):
        s = pa_ref[...] + pb_ref[...] + y_ref[...]
        cnt = ca_ref[...] + cb_ref[...] + 1.0
        if C == 16:
            cw = cnt
        else:
            cw = jnp.broadcast_to(cnt[:, :1], (br, C))
        h = jnp.maximum(s / cw + b_ref[...], 0.0)
        o_ref[...] = jnp.dot(h, w_ref[...], preferred_element_type=F32)

    return pl.pallas_call(
        body,
        grid=(Np // br,),
        in_specs=[pl.BlockSpec((br, C), lambda i: (i, 0)),
                  pl.BlockSpec((br, C), lambda i: (i, 0)),
                  pl.BlockSpec((br, 16), lambda i: (i, 0)),
                  pl.BlockSpec((br, 16), lambda i: (i, 0)),
                  pl.BlockSpec((br, C), lambda i: (i, 0)),
                  pl.BlockSpec((1, C), lambda i: (0, 0)),
                  pl.BlockSpec((C, C2), lambda i: (0, 0))],
        out_specs=pl.BlockSpec((br, C2), lambda i: (i, 0)),
        out_shape=jax.ShapeDtypeStruct((Np, C2), F32),
    )(pa, pb, ca, cb, y, b, W)


def _tc_head(pa, pb, ca, cb, y, b, lw1, lb1, lw2, lb2, lw3, lb3, lw4, lb4,
             ow, ob, br=2048):
    Np, C = y.shape

    def body(pa_ref, pb_ref, ca_ref, cb_ref, y_ref, b_ref,
             w1, v1, w2, v2, w3, v3, w4, v4, wo, vo, o_ref):
        s = pa_ref[...] + pb_ref[...] + y_ref[...]
        cnt = ca_ref[...] + cb_ref[...] + 1.0
        cw = jnp.broadcast_to(cnt[:, :1], (br, C))
        h = jnp.maximum(s / cw + b_ref[...], 0.0)
        h = jnp.maximum(jnp.dot(h, w1[...], preferred_element_type=F32) + v1[...], 0.0)
        h = jnp.maximum(jnp.dot(h, w2[...], preferred_element_type=F32) + v2[...], 0.0)
        h = jnp.maximum(jnp.dot(h, w3[...], preferred_element_type=F32) + v3[...], 0.0)
        h = jnp.maximum(jnp.dot(h, w4[...], preferred_element_type=F32) + v4[...], 0.0)
        o = jnp.dot(h, wo[...], preferred_element_type=F32) + vo[...]
        o_ref[...] = jax.nn.sigmoid(o)

    def mat(a):
        return pl.BlockSpec(a.shape, lambda i: (0, 0))

    return pl.pallas_call(
        body,
        grid=(Np // br,),
        in_specs=[pl.BlockSpec((br, C), lambda i: (i, 0)),
                  pl.BlockSpec((br, C), lambda i: (i, 0)),
                  pl.BlockSpec((br, 16), lambda i: (i, 0)),
                  pl.BlockSpec((br, 16), lambda i: (i, 0)),
                  pl.BlockSpec((br, C), lambda i: (i, 0)),
                  pl.BlockSpec((1, C), lambda i: (0, 0)),
                  mat(lw1), mat(lb1), mat(lw2), mat(lb2),
                  mat(lw3), mat(lb3), mat(lw4), mat(lb4),
                  mat(ow), mat(ob)],
        out_specs=pl.BlockSpec((br, 1), lambda i: (i, 0)),
        out_shape=jax.ShapeDtypeStruct((Np, 1), F32),
    )(pa, pb, ca, cb, y, b, lw1, lb1, lw2, lb2, lw3, lb3, lw4, lb4, ow, ob)


def _sc_body(with_cnt, *refs):
    if with_cnt:
        (y_ref, src_ref, dst_ref, out_ref, cnt_out_ref,
         acc_sh, cnt_sh, src_v, dst_v, rows_v, zero_v, ones_v,
         *sems_g) = refs
    else:
        (y_ref, src_ref, dst_ref, out_ref,
         acc_sh, src_v, dst_v, rows_v, zero_v, *sems_g) = refs

    Np, C = acc_sh.shape
    NCH = src_v.shape[0] - 2
    rt = Np // NS
    c = lax.axis_index("c")
    s = lax.axis_index("s")
    wid = c * NS + s

    z16 = jnp.zeros((16,), F32)

    def zrow(i, _):
        for k in range(C // 16):
            zero_v[i, pl.ds(k * 16, 16)] = z16
        return 0

    lax.fori_loop(0, CHUNK, zrow, 0)
    if with_cnt:
        o16 = jnp.ones((16,), F32)

        def orow(i, _):
            ones_v[i, pl.ds(0, 16)] = o16
            return 0

        lax.fori_loop(0, CHUNK, orow, 0)

    for q in range(rt // CHUNK):
        off = s * rt + q * CHUNK
        pltpu.sync_copy(zero_v, acc_sh.at[pl.ds(off, CHUNK)])
        if with_cnt:
            pltpu.sync_copy(zero_v, cnt_sh.at[pl.ds(off, CHUNK)])
    plsc.subcore_barrier()

    pltpu.sync_copy(src_ref.at[wid], src_v.at[pl.ds(0, NCH)])
    pltpu.sync_copy(dst_ref.at[wid], dst_v.at[pl.ds(0, NCH)])
    sent16 = (Np - 16) + lax.iota(jnp.int32, 16)
    for p in range(2):
        for k in range(CHUNK // 16):
            src_v[NCH + p, pl.ds(k * 16, 16)] = sent16

    def g_start(j, b):
        pltpu.async_copy(y_ref.at[src_v.at[j]], rows_v.at[b], sems_g[b])

    def g_wait(j, b):
        pltpu.make_async_copy(y_ref.at[src_v.at[j]], rows_v.at[b],
                              sems_g[b]).wait()

    g_start(0, 0)

    def chunk_body(j2, _):
        for b in range(2):
            j = 2 * j2 + b
            g_start(j + 1, 1 - b)
            g_wait(j, b)
            pltpu.sync_copy(rows_v.at[b], acc_sh.at[dst_v.at[j]], add=True)
            if with_cnt:
                pltpu.sync_copy(ones_v, cnt_sh.at[dst_v.at[j]], add=True)
        return 0

    lax.fori_loop(0, NCH // 2, chunk_body, 0)
    g_wait(NCH, 0)
    plsc.subcore_barrier()

    pltpu.sync_copy(acc_sh.at[pl.ds(s * rt, rt)], out_ref.at[c, pl.ds(s * rt, rt)])
    if with_cnt:
        pltpu.sync_copy(cnt_sh.at[pl.ds(s * rt, rt)],
                        cnt_out_ref.at[c, pl.ds(s * rt, rt)])


def _sc_aggregate(y, src3, dst3, with_cnt):
    Np, C = y.shape
    NCH = src3.shape[1]
    mesh = plsc.VectorSubcoreMesh(core_axis_name="c", subcore_axis_name="s",
                                  num_cores=NC, num_subcores=NS)
    out_type = [jax.ShapeDtypeStruct((NC, Np, C), F32)]
    assert NCH % 4 == 0 and NCH >= 8
    scratch = [
        pltpu.VMEM_SHARED((Np, C), F32),
        pltpu.VMEM((NCH + 2, CHUNK), jnp.int32),
        pltpu.VMEM((NCH + 2, CHUNK), jnp.int32),
        pltpu.VMEM((2, CHUNK, C), F32),
        pltpu.VMEM((CHUNK, C), F32),
    ] + [pltpu.SemaphoreType.DMA] * 2
    if with_cnt:
        out_type.append(jax.ShapeDtypeStruct((NC, Np, 16), F32))
        scratch.insert(1, pltpu.VMEM_SHARED((Np, 16), F32))
        scratch.insert(6, pltpu.VMEM((CHUNK, 16), F32))
    fn = pl.kernel(functools.partial(_sc_body, with_cnt),
                   out_type=tuple(out_type), mesh=mesh,
                   scratch_types=tuple(scratch),
                   compiler_params=pltpu.CompilerParams(use_tc_tiling_on_sc=False))
    res = fn(y, src3, dst3)
    return res if isinstance(res, (tuple, list)) else (res,)


def kernel(x, edge_index, W1, u1, c1, b1, W2, u2, c2, b2, W3, u3, c3, b3,
           lw1, lb1, lw2, lb2, lw3, lb3, lw4, lb4, ow, ob):
    N, D = x.shape
    E = edge_index.shape[1]
    Np = 10240
    epw = -(-E // NW)
    nch = -(-epw // CHUNK)
    nch = -(-nch // 4) * 4
    ep = NW * nch * CHUNK

    xpad = jnp.zeros((Np, D), F32).at[:N].set(x)
    pad = (Np - CHUNK) + (jnp.arange(ep, dtype=jnp.int32) % CHUNK)
    src3 = pad.at[:E].set(edge_index[0]).reshape(NW, nch, CHUNK)
    dst3 = pad.at[:E].set(edge_index[1]).reshape(NW, nch, CHUNK)

    b1r = b1.reshape(1, -1)
    b2r = b2.reshape(1, -1)
    b3r = b3.reshape(1, -1)

    y1 = _tc_matmul(xpad, W1)
    s1, cnt = _sc_aggregate(y1, src3, dst3, with_cnt=True)
    y2 = _tc_norm_matmul(s1[0], s1[1], cnt[0], cnt[1], y1, b1r, W2)
    (s2,) = _sc_aggregate(y2, src3, dst3, with_cnt=False)
    y3 = _tc_norm_matmul(s2[0], s2[1], cnt[0], cnt[1], y2, b2r, W3)
    (s3,) = _sc_aggregate(y3, src3, dst3, with_cnt=False)
    out = _tc_head(s3[0], s3[1], cnt[0], cnt[1], y3, b3r,
                   lw1, lb1.reshape(1, -1), lw2, lb2.reshape(1, -1),
                   lw3, lb3.reshape(1, -1), lw4, lb4.reshape(1, -1),
                   ow, ob.reshape(1, -1))
    return out[:N]


# speedup vs baseline: 2.5741x; 1.0199x over previous
import functools

import jax
import jax.numpy as jnp
from jax import lax
from jax.experimental import pallas as pl
from jax.experimental.pallas import tpu as pltpu
from jax.experimental.pallas import tpu_sc as plsc

F32 = jnp.float32
NC, NS = 2, 16
NW = NC * NS
CHUNK = 128


def _tc_matmul(xp, W, br=2048):
    Np, D = xp.shape
    C = W.shape[1]

    def body(x_ref, w_ref, o_ref):
        o_ref[...] = jnp.dot(x_ref[...], w_ref[...], preferred_element_type=F32)

    return pl.pallas_call(
        body,
        grid=(Np // br,),
        in_specs=[pl.BlockSpec((br, D), lambda i: (i, 0)),
                  pl.BlockSpec((D, C), lambda i: (0, 0))],
        out_specs=pl.BlockSpec((br, C), lambda i: (i, 0)),
        out_shape=jax.ShapeDtypeStruct((Np, C), F32),
    )(xp, W)


def _tc_norm_matmul(pa, pb, ca, cb, y, b, W, br=2048):
    Np, C = y.shape
    C2 = W.shape[1]

    def body(pa_ref, pb_ref, ca_ref, cb_ref, y_ref, b_ref, w_ref, o_ref):
        s = pa_ref[...] + pb_ref[...] + y_ref[...]
        cnt = ca_ref[...] + cb_ref[...] + 1.0
        if C == 16:
            cw = cnt
        else:
            cw = jnp.broadcast_to(cnt[:, :1], (br, C))
        h = jnp.maximum(s / cw + b_ref[...], 0.0)
        o_ref[...] = jnp.dot(h, w_ref[...], preferred_element_type=F32)

    return pl.pallas_call(
        body,
        grid=(Np // br,),
        in_specs=[pl.BlockSpec((br, C), lambda i: (i, 0)),
                  pl.BlockSpec((br, C), lambda i: (i, 0)),
                  pl.BlockSpec((br, 16), lambda i: (i, 0)),
                  pl.BlockSpec((br, 16), lambda i: (i, 0)),
                  pl.BlockSpec((br, C), lambda i: (i, 0)),
                  pl.BlockSpec((1, C), lambda i: (0, 0)),
                  pl.BlockSpec((C, C2), lambda i: (0, 0))],
        out_specs=pl.BlockSpec((br, C2), lambda i: (i, 0)),
        out_shape=jax.ShapeDtypeStruct((Np, C2), F32),
    )(pa, pb, ca, cb, y, b, W)


def _tc_head(pa, pb, ca, cb, y, b, lw1, lb1, lw2, lb2, lw3, lb3, lw4, lb4,
             ow, ob, br=2048):
    Np, C = y.shape

    def body(pa_ref, pb_ref, ca_ref, cb_ref, y_ref, b_ref,
             w1, v1, w2, v2, w3, v3, w4, v4, wo, vo, o_ref):
        s = pa_ref[...] + pb_ref[...] + y_ref[...]
        cnt = ca_ref[...] + cb_ref[...] + 1.0
        cw = jnp.broadcast_to(cnt[:, :1], (br, C))
        h = jnp.maximum(s / cw + b_ref[...], 0.0)
        h = jnp.maximum(jnp.dot(h, w1[...], preferred_element_type=F32) + v1[...], 0.0)
        h = jnp.maximum(jnp.dot(h, w2[...], preferred_element_type=F32) + v2[...], 0.0)
        h = jnp.maximum(jnp.dot(h, w3[...], preferred_element_type=F32) + v3[...], 0.0)
        h = jnp.maximum(jnp.dot(h, w4[...], preferred_element_type=F32) + v4[...], 0.0)
        o = jnp.dot(h, wo[...], preferred_element_type=F32) + vo[...]
        o_ref[...] = jax.nn.sigmoid(o)

    def mat(a):
        return pl.BlockSpec(a.shape, lambda i: (0, 0))

    return pl.pallas_call(
        body,
        grid=(Np // br,),
        in_specs=[pl.BlockSpec((br, C), lambda i: (i, 0)),
                  pl.BlockSpec((br, C), lambda i: (i, 0)),
                  pl.BlockSpec((br, 16), lambda i: (i, 0)),
                  pl.BlockSpec((br, 16), lambda i: (i, 0)),
                  pl.BlockSpec((br, C), lambda i: (i, 0)),
                  pl.BlockSpec((1, C), lambda i: (0, 0)),
                  mat(lw1), mat(lb1), mat(lw2), mat(lb2),
                  mat(lw3), mat(lb3), mat(lw4), mat(lb4),
                  mat(ow), mat(ob)],
        out_specs=pl.BlockSpec((br, 1), lambda i: (i, 0)),
        out_shape=jax.ShapeDtypeStruct((Np, 1), F32),
    )(pa, pb, ca, cb, y, b, lw1, lb1, lw2, lb2, lw3, lb3, lw4, lb4, ow, ob)


def _sc_body(with_cnt, *refs):
    if with_cnt:
        (y_ref, src_ref, dst_ref, out_ref, cnt_out_ref,
         acc_sh, cnt_sh, src_v, dst_v, rows_v, zero_v, ones_v,
         *sems_g) = refs
    else:
        (y_ref, src_ref, dst_ref, out_ref,
         acc_sh, src_v, dst_v, rows_v, zero_v, *sems_g) = refs

    Np, C = acc_sh.shape
    NCH = src_v.shape[0] - 2
    rt = Np // NS
    c = lax.axis_index("c")
    s = lax.axis_index("s")
    wid = c * NS + s

    z16 = jnp.zeros((16,), F32)

    def zrow(i, _):
        for k in range(C // 16):
            zero_v[i, pl.ds(k * 16, 16)] = z16
        return 0

    lax.fori_loop(0, CHUNK, zrow, 0)
    if with_cnt:
        o16 = jnp.ones((16,), F32)

        def orow(i, _):
            ones_v[i, pl.ds(0, 16)] = o16
            return 0

        lax.fori_loop(0, CHUNK, orow, 0)

    for q in range(rt // CHUNK):
        off = s * rt + q * CHUNK
        pltpu.sync_copy(zero_v, acc_sh.at[pl.ds(off, CHUNK)])
        if with_cnt:
            pltpu.sync_copy(zero_v, cnt_sh.at[pl.ds(off, CHUNK)])
    plsc.subcore_barrier()

    pltpu.sync_copy(src_ref.at[wid], src_v.at[pl.ds(0, NCH)])
    pltpu.sync_copy(dst_ref.at[wid], dst_v.at[pl.ds(0, NCH)])
    sent16 = (Np - 16) + lax.iota(jnp.int32, 16)
    for p in range(2):
        for k in range(CHUNK // 16):
            src_v[NCH + p, pl.ds(k * 16, 16)] = sent16

    def g_start(j, b):
        pltpu.async_copy(y_ref.at[src_v.at[j]], rows_v.at[b], sems_g[b])

    def g_wait(j, b):
        pltpu.make_async_copy(y_ref.at[src_v.at[j]], rows_v.at[b],
                              sems_g[b]).wait()

    g_start(0, 0)
    g_start(1, 1)

    def chunk_body(j4, _):
        for b in range(4):
            j = 4 * j4 + b
            g_start(j + 2, (b + 2) % 4)
            g_wait(j, b)
            pltpu.sync_copy(rows_v.at[b], acc_sh.at[dst_v.at[j]], add=True)
            if with_cnt:
                pltpu.sync_copy(ones_v, cnt_sh.at[dst_v.at[j]], add=True)
        return 0

    lax.fori_loop(0, NCH // 4, chunk_body, 0)
    g_wait(NCH, NCH % 4)
    g_wait(NCH + 1, (NCH + 1) % 4)
    plsc.subcore_barrier()

    pltpu.sync_copy(acc_sh.at[pl.ds(s * rt, rt)], out_ref.at[c, pl.ds(s * rt, rt)])
    if with_cnt:
        pltpu.sync_copy(cnt_sh.at[pl.ds(s * rt, rt)],
                        cnt_out_ref.at[c, pl.ds(s * rt, rt)])


def _sc_aggregate(y, src3, dst3, with_cnt):
    Np, C = y.shape
    NCH = src3.shape[1]
    mesh = plsc.VectorSubcoreMesh(core_axis_name="c", subcore_axis_name="s",
                                  num_cores=NC, num_subcores=NS)
    out_type = [jax.ShapeDtypeStruct((NC, Np, C), F32)]
    assert NCH % 4 == 0 and NCH >= 8
    scratch = [
        pltpu.VMEM_SHARED((Np, C), F32),
        pltpu.VMEM((NCH + 2, CHUNK), jnp.int32),
        pltpu.VMEM((NCH + 2, CHUNK), jnp.int32),
        pltpu.VMEM((4, CHUNK, C), F32),
        pltpu.VMEM((CHUNK, C), F32),
    ] + [pltpu.SemaphoreType.DMA] * 4
    if with_cnt:
        out_type.append(jax.ShapeDtypeStruct((NC, Np, 16), F32))
        scratch.insert(1, pltpu.VMEM_SHARED((Np, 16), F32))
        scratch.insert(6, pltpu.VMEM((CHUNK, 16), F32))
    fn = pl.kernel(functools.partial(_sc_body, with_cnt),
                   out_type=tuple(out_type), mesh=mesh,
                   scratch_types=tuple(scratch),
                   compiler_params=pltpu.CompilerParams(use_tc_tiling_on_sc=False))
    res = fn(y, src3, dst3)
    return res if isinstance(res, (tuple, list)) else (res,)


def kernel(x, edge_index, W1, u1, c1, b1, W2, u2, c2, b2, W3, u3, c3, b3,
           lw1, lb1, lw2, lb2, lw3, lb3, lw4, lb4, ow, ob):
    N, D = x.shape
    E = edge_index.shape[1]
    Np = 10240
    epw = -(-E // NW)
    nch = -(-epw // CHUNK)
    nch = -(-nch // 4) * 4
    ep = NW * nch * CHUNK

    xpad = jnp.zeros((Np, D), F32).at[:N].set(x)
    pad = (Np - CHUNK) + (jnp.arange(ep, dtype=jnp.int32) % CHUNK)
    src3 = pad.at[:E].set(edge_index[0]).reshape(NW, nch, CHUNK)
    dst3 = pad.at[:E].set(edge_index[1]).reshape(NW, nch, CHUNK)

    b1r = b1.reshape(1, -1)
    b2r = b2.reshape(1, -1)
    b3r = b3.reshape(1, -1)

    y1 = _tc_matmul(xpad, W1)
    s1, cnt = _sc_aggregate(y1, src3, dst3, with_cnt=True)
    y2 = _tc_norm_matmul(s1[0], s1[1], cnt[0], cnt[1], y1, b1r, W2)
    (s2,) = _sc_aggregate(y2, src3, dst3, with_cnt=False)
    y3 = _tc_norm_matmul(s2[0], s2[1], cnt[0], cnt[1], y2, b2r, W3)
    (s3,) = _sc_aggregate(y3, src3, dst3, with_cnt=False)
    out = _tc_head(s3[0], s3[1], cnt[0], cnt[1], y3, b3r,
                   lw1, lb1.reshape(1, -1), lw2, lb2.reshape(1, -1),
                   lw3, lb3.reshape(1, -1), lw4, lb4.reshape(1, -1),
                   ow, ob.reshape(1, -1))
    return out[:N]
